# baseline (speedup 1.0000x reference)
import functools

import jax
import jax.numpy as jnp
from jax import lax
from jax.experimental import pallas as pl
from jax.experimental.pallas import tpu as pltpu
from jax.experimental.pallas import tpu_sc as plsc

GRP = 128
NSTREAM = 11


def _sc_gather_sum(tab_pack, idx_all, ent, n_tok, es):
    info = plsc.get_sparse_core_info()
    nw = info.num_cores * info.num_subcores
    vwords = tab_pack.shape[0]
    vstride = es // 2 + 1
    esh = es // 2
    n_grp = idx_all.shape[0] // (NSTREAM * GRP)
    grp_per_tile = n_grp // nw
    grp_per_slab = n_tok // GRP
    gwords = NSTREAM * GRP
    mesh = plsc.VectorSubcoreMesh(core_axis_name="c", subcore_axis_name="s")

    @functools.partial(
        pl.kernel,
        mesh=mesh,
        out_type=jax.ShapeDtypeStruct((n_tok // GRP, ent, es, GRP),
                                      jnp.float32),
        scratch_types=[
            pltpu.VMEM((vwords,), jnp.int32),
            pltpu.VMEM((gwords,), jnp.int32),
            pltpu.VMEM((es, GRP), jnp.float32),
        ],
        compiler_params=pltpu.CompilerParams(needs_layout_passes=False),
    )
    def k(tab_hbm, idx_hbm, out_hbm, tab_v, idx_v, acc_v):
        wid = lax.axis_index("s") * info.num_cores + lax.axis_index("c")
        pltpu.sync_copy(tab_hbm, tab_v)
        himask = jnp.full((16,), -65536, jnp.int32)

        def group_body(g, carry):
            gg = wid * grp_per_tile + g
            pltpu.sync_copy(idx_hbm.at[pl.ds(gg * gwords, gwords)], idx_v)
            for b in range(GRP // 16):
                idx16 = [
                    idx_v[pl.ds(t * GRP + b * 16, 16)] * vstride
                    for t in range(NSTREAM)
                ]
                sl = pl.ds(b * 16, 16)

                def w_body(w, c2):
                    wv = jnp.zeros((16,), jnp.int32) + w
                    alo = jnp.zeros((16,), jnp.float32)
                    ahi = jnp.zeros((16,), jnp.float32)
                    for t in range(NSTREAM):
                        pk = plsc.load_gather(tab_v, [idx16[t] + wv])
                        alo = alo + plsc.bitcast(
                            lax.shift_left(pk, 16), jnp.float32)
                        ahi = ahi + plsc.bitcast(
                            lax.bitwise_and(pk, himask), jnp.float32)
                    acc_v[2 * w, sl] = alo
                    acc_v[2 * w + 1, sl] = ahi
                    return c2

                lax.fori_loop(0, esh, w_body, 0)
            slab = gg // grp_per_slab
            colb = gg % grp_per_slab
            pltpu.sync_copy(acc_v, out_hbm.at[colb, slab])
            return carry

        lax.fori_loop(0, grp_per_tile, group_body, 0)

    return k(tab_pack, idx_all)


def _tc_dense(e3, hp_t, hp_w_c, hp_b_c, ws_t, wp_t, units_w, units_b_c,
              tv1_w, tv1_b_c, tv2_w, tv2_b_c, n_tok, ent, es, vs, r_blk):
    cdims = (((0,), (0,)), ((), ()))

    def body(e_ref, hp_ref, hw_ref, hb_ref, ws_ref, wp_ref, uw_ref, ub_ref,
             t1w_ref, t1b_ref, t2w_ref, t2b_ref, out_ref):
        hw = hw_ref[...]
        hb = hb_ref[...]
        uw = uw_ref[...]
        ub = ub_ref[...]
        t1w = t1w_ref[...]
        h = None
        for k in range(ent):
            side = 1 if k >= 12 else 0
            pub = 1 if k >= 6 else 0
            ck = ws_ref[:, side:side + 1] + wp_ref[:, pub:pub + 1] + hb
            x = e_ref[0, k] + hp_ref[k:k + 1, :] * hw + ck
            x = jnp.maximum(x, 0.0)
            u = lax.dot_general(uw, x, cdims,
                                preferred_element_type=jnp.float32) + ub
            y = lax.dot_general(t1w, jnp.maximum(u, 0.0), cdims,
                                preferred_element_type=jnp.float32)
            h = y if h is None else h + y
        h = h * (1.0 / ent) + t1b_ref[...]
        o = lax.dot_general(t2w_ref[...], jnp.maximum(h, 0.0), cdims,
                            preferred_element_type=jnp.float32) + t2b_ref[...]
        out_ref[...] = o.T

    cfix = lambda shape: pl.BlockSpec(shape, lambda i: tuple(0 for _ in shape))
    return pl.pallas_call(
        body,
        grid=(n_tok // r_blk,),
        in_specs=[
            pl.BlockSpec((1, ent, es, r_blk), lambda i: (i, 0, 0, 0)),
            pl.BlockSpec((ent, r_blk), lambda i: (0, i)),
            cfix((es, 1)), cfix((es, 1)), cfix((es, 2)), cfix((es, 2)),
            cfix((es, es)), cfix((es, 1)), cfix((es, vs)), cfix((vs, 1)),
            cfix((vs, vs)), cfix((vs, 1)),
        ],
        out_specs=pl.BlockSpec((r_blk, vs), lambda i: (i, 0)),
        out_shape=jax.ShapeDtypeStruct((n_tok, vs), jnp.float32),
    )(e3, hp_t, hp_w_c, hp_b_c, ws_t, wp_t, units_w, units_b_c,
      tv1_w, tv1_b_c, tv2_w, tv2_b_c)


def kernel(species, items, abilities, moves, hp_bucket, hp, status, active,
           fainted, W_species, W_item, W_ability, W_moves, W_hp, hp_w, hp_b,
           W_status, W_active, W_fainted, W_side, W_public, units_w, units_b,
           tv1_w, tv1_b, tv2_w, tv2_b):
    B, T, ENT = species.shape
    N = B * T
    NE = N * ENT
    ES = W_species.shape[1]
    VS = tv1_w.shape[1]

    tables = [W_species, W_item, W_ability, W_moves, W_hp, W_status,
              W_active, W_fainted]
    offs = []
    o = 0
    for t in tables:
        offs.append(o)
        o += t.shape[0]
    tab = jnp.concatenate(tables, axis=0)
    tab_pack = lax.bitcast_convert_type(
        tab.astype(jnp.bfloat16).reshape(o, ES // 2, 2), jnp.int32)
    tab_pack = jnp.pad(tab_pack, ((0, 0), (0, 1))).reshape(-1)

    def em(x, off):
        return (x.astype(jnp.int32) + jnp.int32(off)).reshape(N, ENT).T

    streams = [
        em(species, offs[0]), em(items, offs[1]), em(abilities, offs[2]),
        em(moves[..., 0], offs[3]), em(moves[..., 1], offs[3]),
        em(moves[..., 2], offs[3]), em(moves[..., 3], offs[3]),
        em(hp_bucket, offs[4]), em(status, offs[5]),
        em(active, offs[6]), em(fainted, offs[7]),
    ]
    idxs = jnp.stack(streams)
    hp_t = hp.reshape(N, ENT).T

    HALVES = 2
    N2 = N // HALVES
    outs = []
    for h in range(HALVES):
        sl = slice(h * N2, (h + 1) * N2)
        n_grp = (ENT * N2) // GRP
        idx_h = (idxs[:, :, sl].reshape(NSTREAM, n_grp, GRP)
                 .transpose(1, 0, 2).reshape(-1))
        e3 = _sc_gather_sum(tab_pack, idx_h, ENT, N2, ES)
        outs.append(_tc_dense(
            e3, hp_t[:, sl], hp_w.T, hp_b.reshape(ES, 1), W_side.T,
            W_public.T, units_w, units_b.reshape(ES, 1), tv1_w,
            tv1_b.reshape(VS, 1), tv2_w, tv2_b.reshape(VS, 1),
            N2, ENT, ES, VS, GRP))
    out = jnp.concatenate(outs, axis=0)
    return out.reshape(B, T, VS)


# speedup vs baseline: 1.0330x; 1.0330x over previous
import functools

import jax
import jax.numpy as jnp
from jax import lax
from jax.experimental import pallas as pl
from jax.experimental.pallas import tpu as pltpu
from jax.experimental.pallas import tpu_sc as plsc

GRP = 128
NSTREAM = 11


def _sc_gather_sum(tab_pack, idx_all, ent, n_tok, es):
    info = plsc.get_sparse_core_info()
    nw = info.num_cores * info.num_subcores
    vwords = tab_pack.shape[0]
    vstride = es // 2 + 1
    esh = es // 2
    n_grp = idx_all.shape[0] // (NSTREAM * GRP)
    grp_per_tile = n_grp // nw
    grp_per_slab = n_tok // GRP
    gwords = NSTREAM * GRP
    mesh = plsc.VectorSubcoreMesh(core_axis_name="c", subcore_axis_name="s")

    @functools.partial(
        pl.kernel,
        mesh=mesh,
        out_type=jax.ShapeDtypeStruct((ent, es, n_tok), jnp.float32),
        scratch_types=[
            pltpu.VMEM((vwords,), jnp.int32),
            pltpu.VMEM((gwords,), jnp.int32),
            pltpu.VMEM((es, GRP), jnp.float32),
        ],
        compiler_params=pltpu.CompilerParams(needs_layout_passes=False),
    )
    def k(tab_hbm, idx_hbm, out_hbm, tab_v, idx_v, acc_v):
        wid = lax.axis_index("s") * info.num_cores + lax.axis_index("c")
        pltpu.sync_copy(tab_hbm, tab_v)
        himask = jnp.full((16,), -65536, jnp.int32)

        def group_body(g, carry):
            gg = wid * grp_per_tile + g
            pltpu.sync_copy(idx_hbm.at[pl.ds(gg * gwords, gwords)], idx_v)
            for b in range(GRP // 16):
                idx16 = [
                    idx_v[pl.ds(t * GRP + b * 16, 16)] * vstride
                    for t in range(NSTREAM)
                ]
                sl = pl.ds(b * 16, 16)

                def w_body(w, c2):
                    wv = jnp.zeros((16,), jnp.int32) + w
                    alo = jnp.zeros((16,), jnp.float32)
                    ahi = jnp.zeros((16,), jnp.float32)
                    for t in range(NSTREAM):
                        pk = plsc.load_gather(tab_v, [idx16[t] + wv])
                        alo = alo + plsc.bitcast(
                            lax.shift_left(pk, 16), jnp.float32)
                        ahi = ahi + plsc.bitcast(
                            lax.bitwise_and(pk, himask), jnp.float32)
                    acc_v[2 * w, sl] = alo
                    acc_v[2 * w + 1, sl] = ahi
                    return c2

                lax.fori_loop(0, esh, w_body, 0)
            slab = gg // grp_per_slab
            col = (gg % grp_per_slab) * GRP
            pltpu.sync_copy(acc_v, out_hbm.at[slab, :, pl.ds(col, GRP)])
            return carry

        lax.fori_loop(0, grp_per_tile, group_body, 0)

    return k(tab_pack, idx_all)


def _tc_dense(e3, hp_t, hp_w_c, hp_b_c, ws_t, wp_t, units_w, units_b_c,
              tv1_w, tv1_b_c, tv2_w, tv2_b_c, n_tok, ent, es, vs, r_blk):
    cdims = (((0,), (0,)), ((), ()))

    def body(e_ref, hp_ref, hw_ref, hb_ref, ws_ref, wp_ref, uw_ref, ub_ref,
             t1w_ref, t1b_ref, t2w_ref, t2b_ref, out_ref):
        hw = hw_ref[...]
        hb = hb_ref[...]
        uw = uw_ref[...]
        ub = ub_ref[...]
        t1w = t1w_ref[...]
        h = None
        for k in range(ent):
            side = 1 if k >= 12 else 0
            pub = 1 if k >= 6 else 0
            ck = ws_ref[:, side:side + 1] + wp_ref[:, pub:pub + 1] + hb
            x = e_ref[k] + hp_ref[k:k + 1, :] * hw + ck
            x = jnp.maximum(x, 0.0)
            u = lax.dot_general(uw, x, cdims,
                                preferred_element_type=jnp.float32) + ub
            y = lax.dot_general(t1w, jnp.maximum(u, 0.0), cdims,
                                preferred_element_type=jnp.float32)
            h = y if h is None else h + y
        h = h * (1.0 / ent) + t1b_ref[...]
        o = lax.dot_general(t2w_ref[...], jnp.maximum(h, 0.0), cdims,
                            preferred_element_type=jnp.float32) + t2b_ref[...]
        out_ref[...] = o.T

    cfix = lambda shape: pl.BlockSpec(shape, lambda i: tuple(0 for _ in shape))
    return pl.pallas_call(
        body,
        grid=(n_tok // r_blk,),
        in_specs=[
            pl.BlockSpec((ent, es, r_blk), lambda i: (0, 0, i)),
            pl.BlockSpec((ent, r_blk), lambda i: (0, i)),
            cfix((es, 1)), cfix((es, 1)), cfix((es, 2)), cfix((es, 2)),
            cfix((es, es)), cfix((es, 1)), cfix((es, vs)), cfix((vs, 1)),
            cfix((vs, vs)), cfix((vs, 1)),
        ],
        out_specs=pl.BlockSpec((r_blk, vs), lambda i: (i, 0)),
        out_shape=jax.ShapeDtypeStruct((n_tok, vs), jnp.float32),
    )(e3, hp_t, hp_w_c, hp_b_c, ws_t, wp_t, units_w, units_b_c,
      tv1_w, tv1_b_c, tv2_w, tv2_b_c)


def kernel(species, items, abilities, moves, hp_bucket, hp, status, active,
           fainted, W_species, W_item, W_ability, W_moves, W_hp, hp_w, hp_b,
           W_status, W_active, W_fainted, W_side, W_public, units_w, units_b,
           tv1_w, tv1_b, tv2_w, tv2_b):
    B, T, ENT = species.shape
    N = B * T
    NE = N * ENT
    ES = W_species.shape[1]
    VS = tv1_w.shape[1]

    tables = [W_species, W_item, W_ability, W_moves, W_hp, W_status,
              W_active, W_fainted]
    offs = []
    o = 0
    for t in tables:
        offs.append(o)
        o += t.shape[0]
    tab = jnp.concatenate(tables, axis=0)
    tab_pack = lax.bitcast_convert_type(
        tab.astype(jnp.bfloat16).reshape(o, ES // 2, 2), jnp.int32)
    tab_pack = jnp.pad(tab_pack, ((0, 0), (0, 1))).reshape(-1)

    def em(x, off):
        return (x.astype(jnp.int32) + jnp.int32(off)).reshape(N, ENT).T

    streams = [
        em(species, offs[0]), em(items, offs[1]), em(abilities, offs[2]),
        em(moves[..., 0], offs[3]), em(moves[..., 1], offs[3]),
        em(moves[..., 2], offs[3]), em(moves[..., 3], offs[3]),
        em(hp_bucket, offs[4]), em(status, offs[5]),
        em(active, offs[6]), em(fainted, offs[7]),
    ]
    idxs = jnp.stack(streams)
    hp_t = hp.reshape(N, ENT).T

    HALVES = 2
    N2 = N // HALVES
    outs = []
    for h in range(HALVES):
        sl = slice(h * N2, (h + 1) * N2)
        n_grp = (ENT * N2) // GRP
        idx_h = (idxs[:, :, sl].reshape(NSTREAM, n_grp, GRP)
                 .transpose(1, 0, 2).reshape(-1))
        e3 = _sc_gather_sum(tab_pack, idx_h, ENT, N2, ES)
        outs.append(_tc_dense(
            e3, hp_t[:, sl], hp_w.T, hp_b.reshape(ES, 1), W_side.T,
            W_public.T, units_w, units_b.reshape(ES, 1), tv1_w,
            tv1_b.reshape(VS, 1), tv2_w, tv2_b.reshape(VS, 1),
            N2, ENT, ES, VS, 256))
    out = jnp.concatenate(outs, axis=0)
    return out.reshape(B, T, VS)
